# baseline (speedup 1.0000x reference)
import functools

import jax
import jax.numpy as jnp
from jax import lax
from jax.experimental import pallas as pl
from jax.experimental.pallas import tpu as pltpu
from jax.experimental.pallas import tpu_sc as plsc

_B = 16384
_D = 16
_MU = 3.5
_W = 128

_INFO = plsc.get_sparse_core_info()
_NC = _INFO.num_cores
_NS = _INFO.num_subcores
_L = _INFO.num_lanes
_NW = _NC * _NS
_BPW = _B // _NW
_CHUNK = _BPW // 2
_NBLK = _BPW // _L
_CBLK = _CHUNK // _L


def _svd_bias_body(user_idx, item_idx, eu_w, ei_w, ub_w, ib_w, out_hbm,
                   idx_u, idx_i, rows_u, rows_i, ub_v, ib_v, out_v,
                   s0, s1, s2, s3):
    wid = lax.axis_index("s") * _NC + lax.axis_index("c")
    base = wid * _BPW

    pltpu.sync_copy(user_idx.at[pl.ds(base, _BPW)], idx_u)
    pltpu.sync_copy(item_idx.at[pl.ds(base, _BPW)], idx_i)

    cub = pltpu.async_copy(ub_w.at[idx_u], ub_v, s2)
    cib = pltpu.async_copy(ib_w.at[idx_i], ib_v, s3)

    lanes = lax.iota(jnp.int32, _L)

    for chunk in range(2):
        c0 = chunk * _CHUNK
        cu = pltpu.async_copy(eu_w.at[idx_u.at[pl.ds(c0, _CHUNK)]], rows_u, s0)
        ci = pltpu.async_copy(ei_w.at[idx_i.at[pl.ds(c0, _CHUNK)]], rows_i, s1)
        cu.wait()
        ci.wait()

        def blk(k, carry):
            b0 = k * _L
            row_ids = b0 + lanes
            acc = jnp.zeros((_L,), jnp.float32)
            for c in range(_D):
                col = jnp.full((_L,), c, jnp.int32)
                vu = plsc.load_gather(rows_u, [row_ids, col])
                vi = plsc.load_gather(rows_i, [row_ids, col])
                acc = acc + vu * vi
            out_v[pl.ds(c0 + b0, _L)] = acc
            return carry

        lax.fori_loop(0, _CBLK, blk, 0)

    cub.wait()
    cib.wait()

    def bias_blk(k, carry):
        b0 = k * _L
        out_v[pl.ds(b0, _L)] = (out_v[pl.ds(b0, _L)] + ub_v[pl.ds(b0, _L)]
                                + ib_v[pl.ds(b0, _L)] + _MU)
        return carry

    lax.fori_loop(0, _NBLK, bias_blk, 0)

    pltpu.sync_copy(out_v, out_hbm.at[pl.ds(base, _BPW)])


_svd_bias = functools.partial(
    pl.kernel,
    mesh=plsc.VectorSubcoreMesh(core_axis_name="c", subcore_axis_name="s"),
    compiler_params=pltpu.CompilerParams(needs_layout_passes=False),
    out_type=jax.ShapeDtypeStruct((_B,), jnp.float32),
    scratch_types=[
        pltpu.VMEM((_BPW,), jnp.int32),
        pltpu.VMEM((_BPW,), jnp.int32),
        pltpu.VMEM((_CHUNK, _W), jnp.float32),
        pltpu.VMEM((_CHUNK, _W), jnp.float32),
        pltpu.VMEM((_BPW,), jnp.float32),
        pltpu.VMEM((_BPW,), jnp.float32),
        pltpu.VMEM((_BPW,), jnp.float32),
        pltpu.SemaphoreType.DMA,
        pltpu.SemaphoreType.DMA,
        pltpu.SemaphoreType.DMA,
        pltpu.SemaphoreType.DMA,
    ],
)(_svd_bias_body)


def kernel(user_idx, item_idx, embed_user_w, embed_item_w, user_bias_w, item_bias_w):
    eu_p = jnp.pad(embed_user_w, ((0, 0), (0, _W - _D)))
    ei_p = jnp.pad(embed_item_w, ((0, 0), (0, _W - _D)))
    return _svd_bias(user_idx.astype(jnp.int32), item_idx.astype(jnp.int32),
                     eu_p, ei_p,
                     user_bias_w.reshape(-1), item_bias_w.reshape(-1))


# speedup vs baseline: 1.1112x; 1.1112x over previous
import functools

import jax
import jax.numpy as jnp
from jax import lax
from jax.experimental import pallas as pl
from jax.experimental.pallas import tpu as pltpu
from jax.experimental.pallas import tpu_sc as plsc

_B = 16384
_D = 16
_MU = 3.5

_INFO = plsc.get_sparse_core_info()
_NC = _INFO.num_cores
_NS = _INFO.num_subcores
_L = _INFO.num_lanes
_NW = _NC * _NS
_BPW = _B // _NW
_NBLK = _BPW // _L


def _svd_bias_body(*refs):
    user_idx, item_idx = refs[0], refs[1]
    ucols = refs[2:2 + _D]
    icols = refs[2 + _D:2 + 2 * _D]
    ub_w, ib_w = refs[2 + 2 * _D], refs[3 + 2 * _D]
    out_hbm = refs[4 + 2 * _D]
    (idx_u, idx_i, cols_u, cols_i, ub_v, ib_v, out_v,
     s0, s1, s2, s3) = refs[5 + 2 * _D:]

    wid = lax.axis_index("s") * _NC + lax.axis_index("c")
    base = wid * _BPW

    pltpu.sync_copy(user_idx.at[pl.ds(base, _BPW)], idx_u)
    pltpu.sync_copy(item_idx.at[pl.ds(base, _BPW)], idx_i)

    cub = pltpu.async_copy(ub_w.at[idx_u], ub_v, s2)
    cib = pltpu.async_copy(ib_w.at[idx_i], ib_v, s3)

    copies = []
    for c in range(_D):
        copies.append(pltpu.async_copy(ucols[c].at[idx_u], cols_u.at[c], s0))
        copies.append(pltpu.async_copy(icols[c].at[idx_i], cols_i.at[c], s1))
    for cp in copies:
        cp.wait()

    cub.wait()
    cib.wait()

    def blk(k, carry):
        b0 = k * _L
        acc = cols_u[0, pl.ds(b0, _L)] * cols_i[0, pl.ds(b0, _L)]
        for c in range(1, _D):
            acc = acc + cols_u[c, pl.ds(b0, _L)] * cols_i[c, pl.ds(b0, _L)]
        out_v[pl.ds(b0, _L)] = (acc + ub_v[pl.ds(b0, _L)]
                                + ib_v[pl.ds(b0, _L)] + _MU)
        return carry

    lax.fori_loop(0, _NBLK, blk, 0)

    pltpu.sync_copy(out_v, out_hbm.at[pl.ds(base, _BPW)])


_svd_bias = functools.partial(
    pl.kernel,
    mesh=plsc.VectorSubcoreMesh(core_axis_name="c", subcore_axis_name="s"),
    compiler_params=pltpu.CompilerParams(use_tc_tiling_on_sc=False,
                                         needs_layout_passes=False),
    out_type=jax.ShapeDtypeStruct((_B,), jnp.float32),
    scratch_types=[
        pltpu.VMEM((_BPW,), jnp.int32),
        pltpu.VMEM((_BPW,), jnp.int32),
        pltpu.VMEM((_D, _BPW), jnp.float32),
        pltpu.VMEM((_D, _BPW), jnp.float32),
        pltpu.VMEM((_BPW,), jnp.float32),
        pltpu.VMEM((_BPW,), jnp.float32),
        pltpu.VMEM((_BPW,), jnp.float32),
        pltpu.SemaphoreType.DMA,
        pltpu.SemaphoreType.DMA,
        pltpu.SemaphoreType.DMA,
        pltpu.SemaphoreType.DMA,
    ],
)(_svd_bias_body)


def kernel(user_idx, item_idx, embed_user_w, embed_item_w, user_bias_w, item_bias_w):
    ucols = [embed_user_w[:, c] for c in range(_D)]
    icols = [embed_item_w[:, c] for c in range(_D)]
    return _svd_bias(user_idx.astype(jnp.int32), item_idx.astype(jnp.int32),
                     *ucols, *icols,
                     user_bias_w.reshape(-1), item_bias_w.reshape(-1))
